# Initial kernel scaffold; baseline (speedup 1.0000x reference)
import jax
import jax.numpy as jnp
from jax.experimental import pallas as pl


def kernel(classes_idx, weights):
    raise NotImplementedError("write your pallas kernel here")


# speedup vs baseline: 2.3587x; 2.3587x over previous
import jax
import jax.numpy as jnp
from jax import lax
from jax.experimental import pallas as pl
from jax.experimental.pallas import tpu as pltpu
from jax.experimental.pallas import tpu_sc as plsc

_NUM_CORES = 2
_NUM_SUBCORES = 16
_NUM_WORKERS = _NUM_CORES * _NUM_SUBCORES
_WINDOW = 128


def kernel(classes_idx, weights):
    batch, hist = classes_idx.shape
    vocab, embed_dim = weights.shape
    num_indices = batch * hist
    idx_flat = classes_idx.reshape(num_indices)

    per_worker = num_indices // _NUM_WORKERS
    steps = per_worker // _WINDOW

    mesh = plsc.VectorSubcoreMesh(core_axis_name="c", subcore_axis_name="s")

    @pl.kernel(
        out_type=jax.ShapeDtypeStruct((num_indices, embed_dim), weights.dtype),
        mesh=mesh,
        compiler_params=pltpu.CompilerParams(use_tc_tiling_on_sc=False),
        scratch_types=[
            pltpu.VMEM((_WINDOW,), jnp.int32),
            pltpu.VMEM((_WINDOW, embed_dim), weights.dtype),
            pltpu.SemaphoreType.DMA,
        ],
    )
    def gather_kernel(table_hbm, idx_hbm, out_hbm, idx_v, rows_v, sem):
        wid = lax.axis_index("s") * _NUM_CORES + lax.axis_index("c")
        worker_base = wid * per_worker

        @pl.loop(0, steps)
        def _(t):
            base = worker_base + t * _WINDOW
            pltpu.sync_copy(idx_hbm.at[pl.ds(base, _WINDOW)], idx_v)
            pltpu.async_copy(table_hbm.at[idx_v], rows_v, sem).wait()
            pltpu.sync_copy(rows_v, out_hbm.at[pl.ds(base, _WINDOW)])

    out = gather_kernel(weights, idx_flat)
    return out.reshape(batch, hist, embed_dim)


# speedup vs baseline: 2.6998x; 1.1446x over previous
import jax
import jax.numpy as jnp
from jax import lax
from jax.experimental import pallas as pl
from jax.experimental.pallas import tpu as pltpu
from jax.experimental.pallas import tpu_sc as plsc

_NUM_CORES = 2
_NUM_SUBCORES = 16
_NUM_WORKERS = _NUM_CORES * _NUM_SUBCORES
_WINDOW = 128
_WINDOWS_PER_SB = 5
_SB = _WINDOW * _WINDOWS_PER_SB


def kernel(classes_idx, weights):
    batch, hist = classes_idx.shape
    vocab, embed_dim = weights.shape
    num_indices = batch * hist
    idx_flat = classes_idx.reshape(num_indices)

    per_worker = num_indices // _NUM_WORKERS
    num_sb = per_worker // _SB
    num_pairs = num_sb // 2

    mesh = plsc.VectorSubcoreMesh(core_axis_name="c", subcore_axis_name="s")

    @pl.kernel(
        out_type=jax.ShapeDtypeStruct((num_indices, embed_dim), weights.dtype),
        mesh=mesh,
        compiler_params=pltpu.CompilerParams(use_tc_tiling_on_sc=False),
        scratch_types=[
            pltpu.VMEM((per_worker,), jnp.int32),
            pltpu.VMEM((_SB, embed_dim), weights.dtype),
            pltpu.VMEM((_SB, embed_dim), weights.dtype),
            pltpu.SemaphoreType.DMA,
            pltpu.SemaphoreType.DMA,
            pltpu.SemaphoreType.DMA,
        ],
    )
    def gather_kernel(table_hbm, idx_hbm, out_hbm, idx_v, buf0, buf1,
                      gsem, wsem0, wsem1):
        wid = lax.axis_index("s") * _NUM_CORES + lax.axis_index("c")
        worker_base = wid * per_worker

        pltpu.sync_copy(idx_hbm.at[pl.ds(worker_base, per_worker)], idx_v)

        def fire_gathers(buf, sb):
            for k in range(_WINDOWS_PER_SB):
                pltpu.async_copy(
                    table_hbm.at[idx_v.at[pl.ds(sb * _SB + k * _WINDOW, _WINDOW)]],
                    buf.at[pl.ds(k * _WINDOW, _WINDOW)],
                    gsem,
                )

        def drain_gathers(buf):
            pltpu.make_async_copy(out_hbm.at[pl.ds(worker_base, _SB)], buf,
                                  gsem).wait()

        def out_slice(sb):
            return out_hbm.at[pl.ds(worker_base + sb * _SB, _SB)]

        def fire_wb(buf, sb, wsem):
            pltpu.async_copy(buf, out_slice(sb), wsem)

        def wait_wb(buf, wsem):
            pltpu.make_async_copy(buf, out_slice(0), wsem).wait()

        fire_gathers(buf0, 0)

        @pl.loop(0, num_pairs)
        def _(p):
            sb0 = 2 * p
            sb1 = sb0 + 1
            drain_gathers(buf0)

            @pl.when(p > 0)
            def _():
                wait_wb(buf1, wsem1)

            fire_gathers(buf1, sb1)
            fire_wb(buf0, sb0, wsem0)
            drain_gathers(buf1)
            wait_wb(buf0, wsem0)

            @pl.when(p < num_pairs - 1)
            def _():
                fire_gathers(buf0, sb0 + 2)

            fire_wb(buf1, sb1, wsem1)

        wait_wb(buf1, wsem1)

    out = gather_kernel(weights, idx_flat)
    return out.reshape(batch, hist, embed_dim)


# speedup vs baseline: 4.2161x; 1.5617x over previous
import jax
import jax.numpy as jnp
from jax import lax
from jax.experimental import pallas as pl
from jax.experimental.pallas import tpu as pltpu
from jax.experimental.pallas import tpu_sc as plsc

_NUM_CORES = 2
_NUM_SUBCORES = 16
_NUM_WORKERS = _NUM_CORES * _NUM_SUBCORES
_SLOT = 56
_LANES = 128
_WINDOW = 2 * _SLOT
_GROUP = 4


def kernel(classes_idx, weights):
    batch, hist = classes_idx.shape
    vocab, embed_dim = weights.shape

    table = jnp.pad(weights, ((0, 0), (0, _LANES - embed_dim)))
    idx1d = jnp.pad(classes_idx, ((0, 0), (0, _SLOT - hist)),
                    mode="edge").reshape(batch * _SLOT)

    batches_per_worker = batch // _NUM_WORKERS
    idx_per_worker = batches_per_worker * _SLOT
    num_groups = batches_per_worker // _GROUP
    num_iters = num_groups // 2

    mesh = plsc.VectorSubcoreMesh(core_axis_name="c", subcore_axis_name="s")

    @pl.kernel(
        out_type=jax.ShapeDtypeStruct((batch, hist, embed_dim), weights.dtype),
        mesh=mesh,
        scratch_types=[
            pltpu.VMEM((idx_per_worker,), jnp.int32),
            pltpu.VMEM((_GROUP * _SLOT, _LANES), weights.dtype),
            pltpu.VMEM((_GROUP * _SLOT, _LANES), weights.dtype),
            pltpu.VMEM((_GROUP, hist, embed_dim), weights.dtype),
            pltpu.SemaphoreType.DMA,
            pltpu.SemaphoreType.DMA,
        ],
    )
    def gather_kernel(table_hbm, idx_hbm, out_hbm, idx_v, slab0, slab1, outv,
                      gsem, wsem):
        wid = lax.axis_index("s") * _NUM_CORES + lax.axis_index("c")
        worker_base = wid * batches_per_worker

        pltpu.sync_copy(idx_hbm.at[pl.ds(wid * idx_per_worker, idx_per_worker)],
                        idx_v)

        def fire_gathers(slab, g):
            for t in range(2):
                pltpu.async_copy(
                    table_hbm.at[idx_v.at[pl.ds(g * _GROUP * _SLOT + t * _WINDOW,
                                                _WINDOW)]],
                    slab.at[pl.ds(t * _WINDOW, _WINDOW)],
                    gsem,
                )

        def drain_gathers(slab):
            for t in range(2):
                pltpu.make_async_copy(
                    table_hbm.at[pl.ds(0, _WINDOW)],
                    slab.at[pl.ds(t * _WINDOW, _WINDOW)],
                    gsem,
                ).wait()

        def compact(slab):
            @pl.loop(0, hist)
            def _(h):
                for j in range(_GROUP):
                    for k in range(embed_dim // 16):
                        outv.at[j, h, pl.ds(16 * k, 16)][...] = (
                            slab.at[j * _SLOT + h, pl.ds(16 * k, 16)][...])

        def fire_wb(g):
            pltpu.async_copy(
                outv, out_hbm.at[pl.ds(worker_base + g * _GROUP, _GROUP), :, :],
                wsem)

        def wait_wb():
            pltpu.make_async_copy(
                outv, out_hbm.at[pl.ds(worker_base, _GROUP), :, :],
                wsem).wait()

        fire_gathers(slab0, 0)

        @pl.loop(0, num_iters)
        def _(i):
            g0 = 2 * i
            g1 = g0 + 1

            drain_gathers(slab0)
            fire_gathers(slab1, g1)

            @pl.when(i > 0)
            def _():
                wait_wb()

            compact(slab0)
            fire_wb(g0)

            @pl.when(i < num_iters - 1)
            def _():
                fire_gathers(slab0, g0 + 2)

            drain_gathers(slab1)
            wait_wb()
            compact(slab1)
            fire_wb(g1)

        wait_wb()

    return gather_kernel(table, idx1d)


# speedup vs baseline: 6.9209x; 1.6415x over previous
import jax
import jax.numpy as jnp
from jax import lax
from jax.experimental import pallas as pl
from jax.experimental.pallas import tpu as pltpu
from jax.experimental.pallas import tpu_sc as plsc

_NUM_CORES = 2
_NUM_SUBCORES = 16
_NUM_WORKERS = _NUM_CORES * _NUM_SUBCORES
_LANES = 128
_BPW = 128


def kernel(classes_idx, weights):
    batch, hist = classes_idx.shape
    vocab, embed_dim = weights.shape
    rows_per_line = _LANES // embed_dim

    table = weights.reshape(vocab // rows_per_line, _LANES)
    idx_t = classes_idx.T

    num_pairs = hist // 2

    mesh = plsc.VectorSubcoreMesh(core_axis_name="c", subcore_axis_name="s")

    @pl.kernel(
        out_type=jax.ShapeDtypeStruct((hist, embed_dim, batch), weights.dtype),
        mesh=mesh,
        compiler_params=pltpu.CompilerParams(needs_layout_passes=False),
        scratch_types=[
            pltpu.VMEM((hist, _BPW), jnp.int32),
            pltpu.VMEM((hist, _BPW), jnp.int32),
            pltpu.VMEM((_BPW, _LANES), weights.dtype),
            pltpu.VMEM((_BPW, _LANES), weights.dtype),
            pltpu.VMEM((1, embed_dim, _BPW), weights.dtype),
            pltpu.VMEM((1, embed_dim, _BPW), weights.dtype),
            pltpu.SemaphoreType.DMA,
            pltpu.SemaphoreType.DMA,
            pltpu.SemaphoreType.DMA,
        ],
    )
    def gather_kernel(table_hbm, idx_hbm, out_hbm, idx_v, q_v, slab0, slab1,
                      outt0, outt1, gsem, wsem0, wsem1):
        wid = lax.axis_index("s") * _NUM_CORES + lax.axis_index("c")
        b0 = wid * _BPW

        pltpu.sync_copy(idx_hbm.at[:, pl.ds(b0, _BPW)], idx_v)

        iota16 = lax.iota(jnp.int32, 16)
        zero16 = jnp.zeros((16,), jnp.int32)
        perms = [(iota16 + j) & 15 for j in range(16)]

        @pl.loop(0, hist)
        def _(h):
            for m in range(_BPW // 16):
                q_v.at[h, pl.ds(16 * m, 16)][...] = (
                    idx_v.at[h, pl.ds(16 * m, 16)][...] >> 2)

        def fire_gather(slab, h):
            pltpu.async_copy(table_hbm.at[q_v.at[h]], slab, gsem)

        def drain_gather(slab):
            pltpu.make_async_copy(table_hbm.at[pl.ds(0, _BPW)], slab,
                                  gsem).wait()

        def transpose(slab, outt, h):
            @pl.loop(0, _BPW // 16)
            def _(kb):
                rows = 16 * kb + iota16
                r32 = (idx_v.at[h, pl.ds(16 * kb, 16)][...] & 3) * embed_dim
                for cg in range(0, embed_dim, 16):
                    for j in range(16):
                        v = plsc.load_gather(slab, [rows, r32 + cg + perms[j]])
                        plsc.store_scatter(outt, [zero16, cg + perms[j], rows],
                                           v)

        def fire_wb(outt, h, wsem):
            pltpu.async_copy(
                outt, out_hbm.at[pl.ds(h, 1), :, pl.ds(b0, _BPW)], wsem)

        def wait_wb(outt, wsem):
            pltpu.make_async_copy(
                outt, out_hbm.at[pl.ds(0, 1), :, pl.ds(b0, _BPW)], wsem).wait()

        fire_gather(slab0, 0)
        fire_gather(slab1, 1)

        @pl.loop(0, num_pairs)
        def _(p):
            h0 = 2 * p
            h1 = h0 + 1

            drain_gather(slab0)

            @pl.when(p > 0)
            def _():
                wait_wb(outt0, wsem0)

            transpose(slab0, outt0, h0)
            fire_wb(outt0, h0, wsem0)

            @pl.when(p < num_pairs - 1)
            def _():
                fire_gather(slab0, h0 + 2)

            drain_gather(slab1)

            @pl.when(p > 0)
            def _():
                wait_wb(outt1, wsem1)

            transpose(slab1, outt1, h1)
            fire_wb(outt1, h1, wsem1)

            @pl.when(p < num_pairs - 1)
            def _():
                fire_gather(slab1, h1 + 2)

        wait_wb(outt0, wsem0)
        wait_wb(outt1, wsem1)

    out = gather_kernel(table, idx_t)
    return jnp.transpose(out, (2, 0, 1))


# speedup vs baseline: 7.1895x; 1.0388x over previous
import jax
import jax.numpy as jnp
from jax import lax
from jax.experimental import pallas as pl
from jax.experimental.pallas import tpu as pltpu
from jax.experimental.pallas import tpu_sc as plsc

_NUM_CORES = 2
_NUM_SUBCORES = 16
_NUM_WORKERS = _NUM_CORES * _NUM_SUBCORES
_LANES = 128
_BPW = 128


def kernel(classes_idx, weights):
    batch, hist = classes_idx.shape
    vocab, embed_dim = weights.shape
    rows_per_line = _LANES // embed_dim

    table = weights.reshape(vocab // rows_per_line, _LANES)
    idx_t = classes_idx.T

    num_groups = (hist + 1) // 2
    num_pairs = (num_groups + 1) // 2

    mesh = plsc.VectorSubcoreMesh(core_axis_name="c", subcore_axis_name="s")

    @pl.kernel(
        out_type=jax.ShapeDtypeStruct((hist, embed_dim, batch), weights.dtype),
        mesh=mesh,
        compiler_params=pltpu.CompilerParams(needs_layout_passes=False),
        scratch_types=[
            pltpu.VMEM((hist, _BPW), jnp.int32),
            pltpu.VMEM((hist, _BPW), jnp.int32),
            pltpu.VMEM((2 * _BPW, _LANES), weights.dtype),
            pltpu.VMEM((2 * _BPW, _LANES), weights.dtype),
            pltpu.VMEM((2, embed_dim, _BPW), weights.dtype),
            pltpu.VMEM((2, embed_dim, _BPW), weights.dtype),
            pltpu.SemaphoreType.DMA,
            pltpu.SemaphoreType.DMA,
            pltpu.SemaphoreType.DMA,
        ],
    )
    def gather_kernel(table_hbm, idx_hbm, out_hbm, idx_v, q_v, slab0, slab1,
                      outt0, outt1, gsem, wsem0, wsem1):
        wid = lax.axis_index("s") * _NUM_CORES + lax.axis_index("c")
        b0 = wid * _BPW

        pltpu.sync_copy(idx_hbm.at[:, pl.ds(b0, _BPW)], idx_v)

        iota16 = lax.iota(jnp.int32, 16)
        zero16 = jnp.zeros((16,), jnp.int32)
        perms = [(iota16 + j) & 15 for j in range(16)]

        @pl.loop(0, hist)
        def _(h):
            for m in range(_BPW // 16):
                q_v.at[h, pl.ds(16 * m, 16)][...] = (
                    idx_v.at[h, pl.ds(16 * m, 16)][...] >> 2)

        def fire_group(slab, g):
            for t in range(2):
                pltpu.async_copy(table_hbm.at[q_v.at[2 * g + t]],
                                 slab.at[pl.ds(t * _BPW, _BPW)], gsem)

        def drain_group(slab):
            for t in range(2):
                pltpu.make_async_copy(
                    table_hbm.at[pl.ds(0, _BPW)],
                    slab.at[pl.ds(t * _BPW, _BPW)], gsem).wait()

        def transpose(slab, outt, g):
            @pl.loop(0, _BPW // 16)
            def _(kb):
                for t in range(2):
                    rows = t * _BPW + 16 * kb + iota16
                    orows = 16 * kb + iota16
                    r32 = (idx_v.at[2 * g + t, pl.ds(16 * kb, 16)][...]
                           & (rows_per_line - 1)) * embed_dim
                    tvec = jnp.full((16,), t, jnp.int32)
                    for cg in range(0, embed_dim, 16):
                        for j in range(16):
                            v = plsc.load_gather(
                                slab, [rows, r32 + cg + perms[j]])
                            plsc.store_scatter(
                                outt, [tvec, cg + perms[j], orows], v)

        def fire_wb(outt, g, wsem):
            pltpu.async_copy(
                outt, out_hbm.at[pl.ds(2 * g, 2), :, pl.ds(b0, _BPW)], wsem)

        def wait_wb(outt, wsem):
            pltpu.make_async_copy(
                outt, out_hbm.at[pl.ds(0, 2), :, pl.ds(b0, _BPW)], wsem).wait()

        fire_group(slab0, 0)
        fire_group(slab1, 1)

        @pl.loop(0, num_pairs)
        def _(p):
            g0 = 2 * p
            g1 = g0 + 1

            drain_group(slab0)

            @pl.when(p > 0)
            def _():
                wait_wb(outt0, wsem0)

            transpose(slab0, outt0, g0)
            fire_wb(outt0, g0, wsem0)

            @pl.when(g0 + 2 < num_groups)
            def _():
                fire_group(slab0, g0 + 2)

            @pl.when(g1 < num_groups)
            def _():
                drain_group(slab1)

                @pl.when(p > 0)
                def _():
                    wait_wb(outt1, wsem1)

                transpose(slab1, outt1, g1)
                fire_wb(outt1, g1, wsem1)

                @pl.when(g1 + 2 < num_groups)
                def _():
                    fire_group(slab1, g1 + 2)

        wait_wb(outt0, wsem0)
        wait_wb(outt1, wsem1)

    out = gather_kernel(table, idx_t)
    return jnp.transpose(out, (2, 0, 1))
